# baseline (speedup 1.0000x reference)
import functools
import jax
import jax.numpy as jnp
from jax.experimental import pallas as pl
from jax.experimental.pallas import tpu as pltpu

N_USERS = 100000
N_ITEMS = 100000
EMB = 64
LAYERS = 2
N_BEH = 3
LAMB = 0.5
REG = 1e-4
B = 4096
E = 500000
U_OFF = N_USERS + 1
N_NODES = (N_USERS + 1) + (N_ITEMS + 1)


def _loss_body(fp, fn, ap0, an0, ap1, an1, reg, out):
    def bpr(p, n):
        d = p - n
        sig = 1.0 / (1.0 + jnp.exp(-d))
        return -jnp.mean(jnp.log(sig + 1e-10))

    rec = bpr(fp[...], fn[...])
    aux = 0.5 * (bpr(ap0[...], an0[...]) + bpr(ap1[...], an1[...]))
    val = rec + LAMB * aux + REG * reg[0, 0]
    out[...] = jnp.full((1, 1), val, dtype=jnp.float32)


def _final_loss(fp, fn, ap0, an0, ap1, an1, emb_reg):
    shape = (32, 128)
    args = [x.reshape(shape) for x in (fp, fn, ap0, an0, ap1, an1)]
    args.append(emb_reg.reshape(1, 1))
    out = pl.pallas_call(
        _loss_body,
        out_shape=jax.ShapeDtypeStruct((1, 1), jnp.float32),
    )(*args)
    return out.reshape(())


def _build_graph(edge):
    u = edge[0]
    it = edge[1] + U_OFF
    src = jnp.concatenate([u, it])
    dst = jnp.concatenate([it, u])
    deg = jnp.bincount(src, length=N_NODES).astype(jnp.float32)
    norm = jax.lax.rsqrt(jnp.maximum(deg[src], 1.0) * jnp.maximum(deg[dst], 1.0))
    return src, dst, norm


def _lightgcn(emb, graph):
    src, dst, norm = graph
    acc = emb
    e = emb
    for _ in range(LAYERS):
        e = jax.ops.segment_sum(e[src] * norm[:, None], dst, num_segments=N_NODES)
        acc = acc + e
    return acc / (LAYERS + 1)


def _prob(e, u, it):
    return jax.nn.relu(jnp.sum(e[u] * e[U_OFF + it], axis=-1))


def kernel(batch_data, edge_index_aux0, edge_index_aux1, edge_index_target, user_emb, item_emb):
    user = batch_data[:, 0, 0]
    p_item = batch_data[:, 0, 1]
    n_item = batch_data[:, 0, 2]
    all_emb = jnp.concatenate([user_emb, item_emb], axis=0)
    aux_edges = [edge_index_aux0, edge_index_aux1]
    tgt_graph = _build_graph(edge_index_target)
    cond_p, cond_n, comb_p, comb_n, aux_p, aux_n = [], [], [], [], [], []
    aux_dots = []
    for idx in range(N_BEH - 1):
        comb_graph = _build_graph(jnp.concatenate([aux_edges[idx], edge_index_target], axis=1))
        comb_e = _lightgcn(all_emb, comb_graph)
        comb_p.append(_prob(comb_e, user, p_item))
        comb_n.append(_prob(comb_e, user, n_item))
        aux_e = _lightgcn(all_emb, _build_graph(aux_edges[idx]))
        aux_p.append(_prob(aux_e, user, p_item))
        aux_n.append(_prob(aux_e, user, n_item))
        cond_e = _lightgcn(aux_e, tgt_graph)
        cond_p.append(_prob(cond_e, user, p_item))
        cond_n.append(_prob(cond_e, user, n_item))
        au = aux_e[batch_data[:, 1 + idx, 0]]
        ap = aux_e[U_OFF + batch_data[:, 1 + idx, 1]]
        an = aux_e[U_OFF + batch_data[:, 1 + idx, 2]]
        aux_dots.append((jnp.sum(au * ap, axis=-1), jnp.sum(au * an, axis=-1)))
    cond_p = jnp.stack(cond_p)
    cond_n = jnp.stack(cond_n)
    comb_p = jnp.stack(comb_p)
    comb_n = jnp.stack(comb_n)
    aux_p = jnp.stack(aux_p)
    aux_n = jnp.stack(aux_n)

    def front(cond, comb, aux):
        return jnp.sum(cond * jnp.sum(comb * aux, axis=0), axis=0)

    fp = front(cond_p, comb_p, aux_p)
    fn = front(cond_n, comb_n, aux_n)
    emb_reg = (jnp.linalg.norm(user_emb) + jnp.linalg.norm(item_emb)) / item_emb.shape[0]
    return _final_loss(fp, fn, aux_dots[0][0], aux_dots[0][1], aux_dots[1][0], aux_dots[1][1], emb_reg)


# speedup vs baseline: 6.2935x; 6.2935x over previous
import functools
import jax
import jax.numpy as jnp
from jax import lax
from jax.experimental import pallas as pl
from jax.experimental.pallas import tpu as pltpu
from jax.experimental.pallas import tpu_sc as plsc

N_USERS = 100000
N_ITEMS = 100000
EMB = 64
LAYERS = 2
N_BEH = 3
LAMB = 0.5
REG = 1e-4
B = 4096
E = 500000
U_OFF = N_USERS + 1
N_NODES = (N_USERS + 1) + (N_ITEMS + 1)

N_SUPER = 5
C_CORE = 20480
C_TOTAL = 2 * C_CORE
N_PAD = N_SUPER * C_TOTAL
TILE = 2048
K = 128
NSUB = 16
DUMP = C_CORE


def _prop_body(src_hbm, dst_hbm, t_hbm, out_hbm,
               wsrc, wldst, src_dma, ldst_dma, src_tile, dst_tile,
               rows, zbuf, acc_sh, gsem, *, m_pad):
    c = lax.axis_index("c")
    s = lax.axis_index("s")
    m_per_sub = m_pad // NSUB
    n_tiles = m_per_sub // TILE
    rows_per_sub = C_CORE // NSUB
    zrows = 256

    zvec = jnp.zeros((16,), jnp.float32)

    def zb(i, _):
        for l in range(EMB // 16):
            zbuf[i, pl.ds(l * 16, 16)] = zvec
        return 0
    lax.fori_loop(0, zrows, zb, 0)

    lane = lax.iota(jnp.int32, 16)

    for sc_i in range(N_SUPER):
        lo = sc_i * C_TOTAL + c * C_CORE

        for z in range(rows_per_sub // zrows):
            pltpu.sync_copy(zbuf, acc_sh.at[pl.ds(s * rows_per_sub + z * zrows, zrows)])

        @pl.when(s == 0)
        def _():
            pltpu.sync_copy(zbuf.at[pl.ds(0, 16)], acc_sh.at[pl.ds(C_CORE, 16)])

        plsc.subcore_barrier()

        def flush(fill):
            for j in range(K // 16):
                src_dma[pl.ds(j * 16, 16)] = wsrc[pl.ds(j * 16, 16)]
                ldst_dma[pl.ds(j * 16, 16)] = wldst[pl.ds(j * 16, 16)]
            pltpu.async_copy(t_hbm.at[src_dma], rows, gsem).wait()
            pltpu.sync_copy(rows, acc_sh.at[ldst_dma], add=True)
            sv = wsrc[pl.ds(K, 16)]
            dv = wldst[pl.ds(K, 16)]
            wsrc[pl.ds(0, 16)] = sv
            wldst[pl.ds(0, 16)] = dv
            return fill - K

        def tile_body(ti, fill):
            base = s * m_per_sub + ti * TILE
            pltpu.sync_copy(src_hbm.at[pl.ds(base, TILE)], src_tile)
            pltpu.sync_copy(dst_hbm.at[pl.ds(base, TILE)], dst_tile)

            def vec_body(vi, fill):
                dv = dst_tile[pl.ds(vi * 16, 16)]
                m = (dv >= lo) & (dv < lo + C_CORE)
                sv = src_tile[pl.ds(vi * 16, 16)]
                mi = m.astype(jnp.int32)
                cs = plsc.cumsum(mi)
                pos = (fill + cs) - mi
                plsc.store_scatter(wsrc, [pos], sv, mask=m)
                plsc.store_scatter(wldst, [pos], dv - lo, mask=m)
                fill = fill + jnp.sum(mi)
                return lax.cond(fill >= K, flush, lambda f: f, fill)

            return lax.fori_loop(0, TILE // 16, vec_body, fill)

        fill = lax.fori_loop(0, n_tiles, tile_body, jnp.int32(0))

        for j in range(K // 16):
            pos = lane + (j * 16)
            pm = pos >= fill
            sv = wsrc[pl.ds(j * 16, 16)]
            dv = wldst[pl.ds(j * 16, 16)]
            wsrc[pl.ds(j * 16, 16)] = jnp.where(pm, lane + s * 16, sv)
            wldst[pl.ds(j * 16, 16)] = jnp.where(pm, lane + DUMP, dv)
        flush(fill)

        plsc.subcore_barrier()

        for z in range(rows_per_sub // zrows):
            r0 = s * rows_per_sub + z * zrows
            pltpu.sync_copy(acc_sh.at[pl.ds(r0, zrows)],
                            out_hbm.at[pl.ds(lo + r0, zrows)])


@functools.cache
def _make_propagate(m_pad):
    mesh = plsc.VectorSubcoreMesh(core_axis_name="c", subcore_axis_name="s")
    body = functools.partial(_prop_body, m_pad=m_pad)
    return pl.kernel(
        body,
        mesh=mesh,
        compiler_params=pltpu.CompilerParams(needs_layout_passes=False, use_tc_tiling_on_sc=False),
        out_type=jax.ShapeDtypeStruct((N_PAD, EMB), jnp.float32),
        scratch_types=[
            pltpu.VMEM((K + 32,), jnp.int32),
            pltpu.VMEM((K + 32,), jnp.int32),
            pltpu.VMEM((K,), jnp.int32),
            pltpu.VMEM((K,), jnp.int32),
            pltpu.VMEM((TILE,), jnp.int32),
            pltpu.VMEM((TILE,), jnp.int32),
            pltpu.VMEM((K, EMB), jnp.float32),
            pltpu.VMEM((256, EMB), jnp.float32),
            pltpu.VMEM_SHARED((C_CORE + 16, EMB), jnp.float32),
            pltpu.SemaphoreType.DMA,
        ],
    )


def _propagate(src, dst, t):
    return _make_propagate(src.shape[0])(src, dst, t)


def _pad_edges(src, dst):
    m = src.shape[0]
    step = NSUB * TILE
    m_pad = ((m + step - 1) // step) * step
    pad = m_pad - m
    if pad:
        fill_idx = (jnp.arange(pad, dtype=jnp.int32) % 16) + (N_PAD - 16)
        src = jnp.concatenate([src, fill_idx])
        dst = jnp.concatenate([dst, fill_idx])
    return src, dst


def _graph(u, it):
    src = jnp.concatenate([u, it + U_OFF])
    dst = jnp.concatenate([it + U_OFF, u])
    deg = jnp.bincount(src, length=N_NODES).astype(jnp.float32)
    invd = lax.rsqrt(jnp.maximum(deg, 1.0))
    invd = jnp.pad(invd, (0, N_PAD - N_NODES))[:, None]
    src, dst = _pad_edges(src, dst)
    return src, dst, invd


def _lightgcn_sc(e0, graph):
    src, dst, invd = graph
    t0 = e0 * invd
    y1 = _propagate(src, dst, t0)
    t1 = y1 * (invd * invd)
    y2 = _propagate(src, dst, t1)
    return (e0 + invd * (y1 + y2)) * (1.0 / (LAYERS + 1))


def _loss_body(fp, fn, ap0, an0, ap1, an1, reg, out):
    def bpr(p, n):
        d = p - n
        sig = 1.0 / (1.0 + jnp.exp(-d))
        return -jnp.mean(jnp.log(sig + 1e-10))

    rec = bpr(fp[...], fn[...])
    aux = 0.5 * (bpr(ap0[...], an0[...]) + bpr(ap1[...], an1[...]))
    val = rec + LAMB * aux + REG * reg[0, 0]
    out[...] = jnp.full((1, 1), val, dtype=jnp.float32)


def _final_loss(fp, fn, ap0, an0, ap1, an1, emb_reg):
    shape = (32, 128)
    args = [x.reshape(shape) for x in (fp, fn, ap0, an0, ap1, an1)]
    args.append(emb_reg.reshape(1, 1))
    out = pl.pallas_call(
        _loss_body,
        out_shape=jax.ShapeDtypeStruct((1, 1), jnp.float32),
    )(*args)
    return out.reshape(())


def _prob(e, u, it):
    return jax.nn.relu(jnp.sum(e[u] * e[U_OFF + it], axis=-1))


def kernel(batch_data, edge_index_aux0, edge_index_aux1, edge_index_target, user_emb, item_emb):
    user = batch_data[:, 0, 0]
    p_item = batch_data[:, 0, 1]
    n_item = batch_data[:, 0, 2]
    all_emb = jnp.concatenate([user_emb, item_emb], axis=0)
    e0 = jnp.pad(all_emb, ((0, N_PAD - N_NODES), (0, 0)))
    aux_edges = [edge_index_aux0, edge_index_aux1]
    tgt_graph = _graph(edge_index_target[0], edge_index_target[1])
    cond_p, cond_n, comb_p, comb_n, aux_p, aux_n = [], [], [], [], [], []
    aux_dots = []
    for idx in range(N_BEH - 1):
        comb_u = jnp.concatenate([aux_edges[idx][0], edge_index_target[0]])
        comb_it = jnp.concatenate([aux_edges[idx][1], edge_index_target[1]])
        comb_e = _lightgcn_sc(e0, _graph(comb_u, comb_it))
        comb_p.append(_prob(comb_e, user, p_item))
        comb_n.append(_prob(comb_e, user, n_item))
        aux_e = _lightgcn_sc(e0, _graph(aux_edges[idx][0], aux_edges[idx][1]))
        aux_p.append(_prob(aux_e, user, p_item))
        aux_n.append(_prob(aux_e, user, n_item))
        cond_e = _lightgcn_sc(aux_e, tgt_graph)
        cond_p.append(_prob(cond_e, user, p_item))
        cond_n.append(_prob(cond_e, user, n_item))
        au = aux_e[batch_data[:, 1 + idx, 0]]
        ap = aux_e[U_OFF + batch_data[:, 1 + idx, 1]]
        an = aux_e[U_OFF + batch_data[:, 1 + idx, 2]]
        aux_dots.append((jnp.sum(au * ap, axis=-1), jnp.sum(au * an, axis=-1)))
    cond_p = jnp.stack(cond_p)
    cond_n = jnp.stack(cond_n)
    comb_p = jnp.stack(comb_p)
    comb_n = jnp.stack(comb_n)
    aux_p = jnp.stack(aux_p)
    aux_n = jnp.stack(aux_n)

    def front(cond, comb, aux):
        return jnp.sum(cond * jnp.sum(comb * aux, axis=0), axis=0)

    fp = front(cond_p, comb_p, aux_p)
    fn = front(cond_n, comb_n, aux_n)
    emb_reg = (jnp.linalg.norm(user_emb) + jnp.linalg.norm(item_emb)) / item_emb.shape[0]
    return _final_loss(fp, fn, aux_dots[0][0], aux_dots[0][1], aux_dots[1][0], aux_dots[1][1], emb_reg)


# speedup vs baseline: 8.9458x; 1.4215x over previous
import functools
import jax
import jax.numpy as jnp
from jax import lax
from jax.experimental import pallas as pl
from jax.experimental.pallas import tpu as pltpu
from jax.experimental.pallas import tpu_sc as plsc

N_USERS = 100000
N_ITEMS = 100000
EMB = 64
LAYERS = 2
N_BEH = 3
LAMB = 0.5
REG = 1e-4
B = 4096
E = 500000
U_OFF = N_USERS + 1
N_NODES = (N_USERS + 1) + (N_ITEMS + 1)

N_SUPER = 5
C_CORE = 20480
C_TOTAL = 2 * C_CORE
N_PAD = N_SUPER * C_TOTAL
TILE = 2048
K = 512
NSTR = 4
NSUB = 16
DUMP = C_CORE


def _prop_body(src_hbm, dst_hbm, t_hbm, out_hbm,
               wsrc, wldst, src_dma, ldst_dma, src_tile, dst_tile,
               rows, zbuf, acc_sh, gsem, ssem, *, m_pad):
    c = lax.axis_index("c")
    s = lax.axis_index("s")
    m_per_sub = m_pad // NSUB
    n_tiles = m_per_sub // TILE
    rows_per_sub = C_CORE // NSUB
    zrows = 64

    zvec = jnp.zeros((16,), jnp.float32)

    def zb(i, _):
        for l in range(EMB // 16):
            zbuf[i, pl.ds(l * 16, 16)] = zvec
        return 0
    lax.fori_loop(0, zrows, zb, 0)

    lane = lax.iota(jnp.int32, 16)

    for sc_i in range(N_SUPER):
        lo = sc_i * C_TOTAL + c * C_CORE

        for z in range(rows_per_sub // zrows):
            pltpu.sync_copy(zbuf, acc_sh.at[pl.ds(s * rows_per_sub + z * zrows, zrows)])

        @pl.when(s == 0)
        def _():
            pltpu.sync_copy(zbuf.at[pl.ds(0, 16)], acc_sh.at[pl.ds(C_CORE, 16)])

        plsc.subcore_barrier()

        def flush(fill):
            for j in range(NSTR):
                for i in range(128 // 16):
                    src_dma[j, pl.ds(i * 16, 16)] = wsrc[pl.ds(j * 128 + i * 16, 16)]
                    ldst_dma[j, pl.ds(i * 16, 16)] = wldst[pl.ds(j * 128 + i * 16, 16)]
            gh = [pltpu.async_copy(t_hbm.at[src_dma.at[j]], rows.at[j], gsem)
                  for j in range(NSTR)]
            sh = []
            for j in range(NSTR):
                gh[j].wait()
                sh.append(pltpu.async_copy(rows.at[j], acc_sh.at[ldst_dma.at[j]],
                                           ssem, add=True))
            for h in sh:
                h.wait()
            sv = wsrc[pl.ds(K, 16)]
            dv = wldst[pl.ds(K, 16)]
            wsrc[pl.ds(0, 16)] = sv
            wldst[pl.ds(0, 16)] = dv
            return fill - K

        def tile_body(ti, fill):
            base = s * m_per_sub + ti * TILE
            pltpu.sync_copy(src_hbm.at[pl.ds(base, TILE)], src_tile)
            pltpu.sync_copy(dst_hbm.at[pl.ds(base, TILE)], dst_tile)

            def vec_body(vi, fill):
                dv = dst_tile[pl.ds(vi * 16, 16)]
                m = (dv >= lo) & (dv < lo + C_CORE)
                sv = src_tile[pl.ds(vi * 16, 16)]
                mi = m.astype(jnp.int32)
                cs = plsc.cumsum(mi)
                pos = (fill + cs) - mi
                plsc.store_scatter(wsrc, [pos], sv, mask=m)
                plsc.store_scatter(wldst, [pos], dv - lo, mask=m)
                fill = fill + jnp.sum(mi)
                return lax.cond(fill >= K, flush, lambda f: f, fill)

            return lax.fori_loop(0, TILE // 16, vec_body, fill)

        fill = lax.fori_loop(0, n_tiles, tile_body, jnp.int32(0))

        for j in range(K // 16):
            pos = lane + (j * 16)
            pm = pos >= fill
            sv = wsrc[pl.ds(j * 16, 16)]
            dv = wldst[pl.ds(j * 16, 16)]
            wsrc[pl.ds(j * 16, 16)] = jnp.where(pm, lane + s * 16, sv)
            wldst[pl.ds(j * 16, 16)] = jnp.where(pm, lane + DUMP, dv)
        flush(fill)

        plsc.subcore_barrier()

        for z in range(rows_per_sub // zrows):
            r0 = s * rows_per_sub + z * zrows
            pltpu.sync_copy(acc_sh.at[pl.ds(r0, zrows)],
                            out_hbm.at[pl.ds(lo + r0, zrows)])


@functools.cache
def _make_propagate(m_pad):
    mesh = plsc.VectorSubcoreMesh(core_axis_name="c", subcore_axis_name="s")
    body = functools.partial(_prop_body, m_pad=m_pad)
    return pl.kernel(
        body,
        mesh=mesh,
        compiler_params=pltpu.CompilerParams(needs_layout_passes=False, use_tc_tiling_on_sc=False),
        out_type=jax.ShapeDtypeStruct((N_PAD, EMB), jnp.float32),
        scratch_types=[
            pltpu.VMEM((K + 32,), jnp.int32),
            pltpu.VMEM((K + 32,), jnp.int32),
            pltpu.VMEM((NSTR, 128), jnp.int32),
            pltpu.VMEM((NSTR, 128), jnp.int32),
            pltpu.VMEM((TILE,), jnp.int32),
            pltpu.VMEM((TILE,), jnp.int32),
            pltpu.VMEM((NSTR, 128, EMB), jnp.float32),
            pltpu.VMEM((64, EMB), jnp.float32),
            pltpu.VMEM_SHARED((C_CORE + 16, EMB), jnp.float32),
            pltpu.SemaphoreType.DMA,
            pltpu.SemaphoreType.DMA,
        ],
    )


def _propagate(src, dst, t):
    return _make_propagate(src.shape[0])(src, dst, t)


def _pad_edges(src, dst):
    m = src.shape[0]
    step = NSUB * TILE
    m_pad = ((m + step - 1) // step) * step
    pad = m_pad - m
    if pad:
        fill_idx = (jnp.arange(pad, dtype=jnp.int32) % 16) + (N_PAD - 16)
        src = jnp.concatenate([src, fill_idx])
        dst = jnp.concatenate([dst, fill_idx])
    return src, dst


def _graph(u, it):
    src = jnp.concatenate([u, it + U_OFF])
    dst = jnp.concatenate([it + U_OFF, u])
    deg = jnp.bincount(src, length=N_NODES).astype(jnp.float32)
    invd = lax.rsqrt(jnp.maximum(deg, 1.0))
    invd = jnp.pad(invd, (0, N_PAD - N_NODES))[:, None]
    src, dst = _pad_edges(src, dst)
    return src, dst, invd


def _lightgcn_sc(e0, graph):
    src, dst, invd = graph
    t0 = e0 * invd
    y1 = _propagate(src, dst, t0)
    t1 = y1 * (invd * invd)
    y2 = _propagate(src, dst, t1)
    return (e0 + invd * (y1 + y2)) * (1.0 / (LAYERS + 1))


def _loss_body(fp, fn, ap0, an0, ap1, an1, reg, out):
    def bpr(p, n):
        d = p - n
        sig = 1.0 / (1.0 + jnp.exp(-d))
        return -jnp.mean(jnp.log(sig + 1e-10))

    rec = bpr(fp[...], fn[...])
    aux = 0.5 * (bpr(ap0[...], an0[...]) + bpr(ap1[...], an1[...]))
    val = rec + LAMB * aux + REG * reg[0, 0]
    out[...] = jnp.full((1, 1), val, dtype=jnp.float32)


def _final_loss(fp, fn, ap0, an0, ap1, an1, emb_reg):
    shape = (32, 128)
    args = [x.reshape(shape) for x in (fp, fn, ap0, an0, ap1, an1)]
    args.append(emb_reg.reshape(1, 1))
    out = pl.pallas_call(
        _loss_body,
        out_shape=jax.ShapeDtypeStruct((1, 1), jnp.float32),
    )(*args)
    return out.reshape(())


def _prob(e, u, it):
    return jax.nn.relu(jnp.sum(e[u] * e[U_OFF + it], axis=-1))


def kernel(batch_data, edge_index_aux0, edge_index_aux1, edge_index_target, user_emb, item_emb):
    user = batch_data[:, 0, 0]
    p_item = batch_data[:, 0, 1]
    n_item = batch_data[:, 0, 2]
    all_emb = jnp.concatenate([user_emb, item_emb], axis=0)
    e0 = jnp.pad(all_emb, ((0, N_PAD - N_NODES), (0, 0)))
    aux_edges = [edge_index_aux0, edge_index_aux1]
    tgt_graph = _graph(edge_index_target[0], edge_index_target[1])
    cond_p, cond_n, comb_p, comb_n, aux_p, aux_n = [], [], [], [], [], []
    aux_dots = []
    for idx in range(N_BEH - 1):
        comb_u = jnp.concatenate([aux_edges[idx][0], edge_index_target[0]])
        comb_it = jnp.concatenate([aux_edges[idx][1], edge_index_target[1]])
        comb_e = _lightgcn_sc(e0, _graph(comb_u, comb_it))
        comb_p.append(_prob(comb_e, user, p_item))
        comb_n.append(_prob(comb_e, user, n_item))
        aux_e = _lightgcn_sc(e0, _graph(aux_edges[idx][0], aux_edges[idx][1]))
        aux_p.append(_prob(aux_e, user, p_item))
        aux_n.append(_prob(aux_e, user, n_item))
        cond_e = _lightgcn_sc(aux_e, tgt_graph)
        cond_p.append(_prob(cond_e, user, p_item))
        cond_n.append(_prob(cond_e, user, n_item))
        au = aux_e[batch_data[:, 1 + idx, 0]]
        ap = aux_e[U_OFF + batch_data[:, 1 + idx, 1]]
        an = aux_e[U_OFF + batch_data[:, 1 + idx, 2]]
        aux_dots.append((jnp.sum(au * ap, axis=-1), jnp.sum(au * an, axis=-1)))
    cond_p = jnp.stack(cond_p)
    cond_n = jnp.stack(cond_n)
    comb_p = jnp.stack(comb_p)
    comb_n = jnp.stack(comb_n)
    aux_p = jnp.stack(aux_p)
    aux_n = jnp.stack(aux_n)

    def front(cond, comb, aux):
        return jnp.sum(cond * jnp.sum(comb * aux, axis=0), axis=0)

    fp = front(cond_p, comb_p, aux_p)
    fn = front(cond_n, comb_n, aux_n)
    emb_reg = (jnp.linalg.norm(user_emb) + jnp.linalg.norm(item_emb)) / item_emb.shape[0]
    return _final_loss(fp, fn, aux_dots[0][0], aux_dots[0][1], aux_dots[1][0], aux_dots[1][1], emb_reg)


# speedup vs baseline: 13.5412x; 1.5137x over previous
import functools
import jax
import jax.numpy as jnp
from jax import lax
from jax.experimental import pallas as pl
from jax.experimental.pallas import tpu as pltpu
from jax.experimental.pallas import tpu_sc as plsc

N_USERS = 100000
N_ITEMS = 100000
EMB = 64
LAYERS = 2
N_BEH = 3
LAMB = 0.5
REG = 1e-4
B = 4096
E = 500000
U_OFF = N_USERS + 1
N_NODES = (N_USERS + 1) + (N_ITEMS + 1)

N_SUPER = 5
C_CORE = 20480
C_TOTAL = 2 * C_CORE
N_PAD = N_SUPER * C_TOTAL
TILE = 2048
K = 512
NSTR = 4
NSUB = 16
DUMP = C_CORE


def _prop_body(src_hbm, dst_hbm, t_hbm, out_hbm,
               wsrc, wldst, src_dma, ldst_dma, src_tile, dst_tile,
               rows, zbuf, acc_sh, gsem, ssem, *, m_pad):
    c = lax.axis_index("c")
    s = lax.axis_index("s")
    m_per_sub = m_pad // NSUB
    n_tiles = m_per_sub // TILE
    rows_per_sub = C_CORE // NSUB
    zrows = 64

    zvec = jnp.zeros((16,), jnp.float32)

    def zb(i, _):
        for l in range(EMB // 16):
            zbuf[i, pl.ds(l * 16, 16)] = zvec
        return 0
    lax.fori_loop(0, zrows, zb, 0)

    lane = lax.iota(jnp.int32, 16)

    for sc_i in range(N_SUPER):
        lo = sc_i * C_TOTAL + c * C_CORE

        for z in range(rows_per_sub // zrows):
            pltpu.sync_copy(zbuf, acc_sh.at[pl.ds(s * rows_per_sub + z * zrows, zrows)])

        @pl.when(s == 0)
        def _():
            pltpu.sync_copy(zbuf.at[pl.ds(0, 16)], acc_sh.at[pl.ds(C_CORE, 16)])

        plsc.subcore_barrier()

        def flush(fill):
            for j in range(NSTR):
                for i in range(128 // 16):
                    src_dma[j, pl.ds(i * 16, 16)] = wsrc[pl.ds(j * 128 + i * 16, 16)]
                    ldst_dma[j, pl.ds(i * 16, 16)] = wldst[pl.ds(j * 128 + i * 16, 16)]
            gh = [pltpu.async_copy(t_hbm.at[src_dma.at[j]], rows.at[j], gsem)
                  for j in range(NSTR)]
            sh = []
            for j in range(NSTR):
                gh[j].wait()
                sh.append(pltpu.async_copy(rows.at[j], acc_sh.at[ldst_dma.at[j]],
                                           ssem, add=True))
            for h in sh:
                h.wait()
            sv = wsrc[pl.ds(K, 16)]
            dv = wldst[pl.ds(K, 16)]
            wsrc[pl.ds(0, 16)] = sv
            wldst[pl.ds(0, 16)] = dv
            return fill - K

        def tile_body(ti, fill):
            base = s * m_per_sub + ti * TILE
            pltpu.sync_copy(src_hbm.at[pl.ds(base, TILE)], src_tile)
            pltpu.sync_copy(dst_hbm.at[pl.ds(base, TILE)], dst_tile)

            def vec_body(vi, fill):
                dv = dst_tile[pl.ds(vi * 16, 16)]
                m = (dv >= lo) & (dv < lo + C_CORE)
                sv = src_tile[pl.ds(vi * 16, 16)]
                mi = m.astype(jnp.int32)
                cs = plsc.cumsum(mi)
                pos = (fill + cs) - mi
                plsc.store_scatter(wsrc, [pos], sv, mask=m)
                plsc.store_scatter(wldst, [pos], dv - lo, mask=m)
                fill = fill + jnp.sum(mi)
                return lax.cond(fill >= K, flush, lambda f: f, fill)

            return lax.fori_loop(0, TILE // 16, vec_body, fill)

        fill = lax.fori_loop(0, n_tiles, tile_body, jnp.int32(0))

        for j in range(K // 16):
            pos = lane + (j * 16)
            pm = pos >= fill
            sv = wsrc[pl.ds(j * 16, 16)]
            dv = wldst[pl.ds(j * 16, 16)]
            wsrc[pl.ds(j * 16, 16)] = jnp.where(pm, lane + s * 16, sv)
            wldst[pl.ds(j * 16, 16)] = jnp.where(pm, lane + DUMP, dv)
        flush(fill)

        plsc.subcore_barrier()

        for z in range(rows_per_sub // zrows):
            r0 = s * rows_per_sub + z * zrows
            pltpu.sync_copy(acc_sh.at[pl.ds(r0, zrows)],
                            out_hbm.at[pl.ds(lo + r0, zrows)])


@functools.cache
def _make_propagate(m_pad):
    mesh = plsc.VectorSubcoreMesh(core_axis_name="c", subcore_axis_name="s")
    body = functools.partial(_prop_body, m_pad=m_pad)
    return pl.kernel(
        body,
        mesh=mesh,
        compiler_params=pltpu.CompilerParams(needs_layout_passes=False, use_tc_tiling_on_sc=False),
        out_type=jax.ShapeDtypeStruct((N_PAD, EMB), jnp.float32),
        scratch_types=[
            pltpu.VMEM((K + 32,), jnp.int32),
            pltpu.VMEM((K + 32,), jnp.int32),
            pltpu.VMEM((NSTR, 128), jnp.int32),
            pltpu.VMEM((NSTR, 128), jnp.int32),
            pltpu.VMEM((TILE,), jnp.int32),
            pltpu.VMEM((TILE,), jnp.int32),
            pltpu.VMEM((NSTR, 128, EMB), jnp.float32),
            pltpu.VMEM((64, EMB), jnp.float32),
            pltpu.VMEM_SHARED((C_CORE + 16, EMB), jnp.float32),
            pltpu.SemaphoreType.DMA,
            pltpu.SemaphoreType.DMA,
        ],
    )


def _propagate(src, dst, t):
    src, dst = _pad_edges(src, dst)
    return _make_propagate(src.shape[0])(src, dst, t)


NSLOT = 3 * B
NSLOT_PAD = 13312
BM_WORDS = N_PAD // 32


def _prop_rows_body(src_hbm, dst_hbm, t_hbm, bm_hbm, pos_hbm, out_hbm,
                    wsrc, wdst, src_dma, dst_dma, posval, src_tile, dst_tile,
                    rows, zbuf, bitmap, racc_sh, gsem, psem, ssem, *, m_pad):
    c = lax.axis_index("c")
    s = lax.axis_index("s")
    w = c * NSUB + s
    m_per_w = m_pad // (2 * NSUB)
    n_tiles = m_per_w // TILE
    rows_per_sub = NSLOT_PAD // NSUB

    zvec = jnp.zeros((16,), jnp.float32)

    def zb(i, _):
        for l in range(EMB // 16):
            zbuf[i, pl.ds(l * 16, 16)] = zvec
        return 0
    lax.fori_loop(0, 64, zb, 0)

    pltpu.sync_copy(bm_hbm, bitmap)
    for z in range(rows_per_sub // 64):
        pltpu.sync_copy(zbuf, racc_sh.at[pl.ds(s * rows_per_sub + z * 64, 64)])
    plsc.subcore_barrier()

    lane = lax.iota(jnp.int32, 16)

    def flush(fill):
        for j in range(NSTR):
            for i in range(128 // 16):
                src_dma[j, pl.ds(i * 16, 16)] = wsrc[pl.ds(j * 128 + i * 16, 16)]
                dst_dma[j, pl.ds(i * 16, 16)] = wdst[pl.ds(j * 128 + i * 16, 16)]
        gh = [pltpu.async_copy(t_hbm.at[src_dma.at[j]], rows.at[j], gsem)
              for j in range(NSTR)]
        ph = [pltpu.async_copy(pos_hbm.at[dst_dma.at[j]], posval.at[j], psem)
              for j in range(NSTR)]
        sh = []
        for j in range(NSTR):
            gh[j].wait()
            ph[j].wait()
            sh.append(pltpu.async_copy(rows.at[j], racc_sh.at[posval.at[j]],
                                       ssem, add=True))
        for h in sh:
            h.wait()
        sv = wsrc[pl.ds(K, 16)]
        dv = wdst[pl.ds(K, 16)]
        wsrc[pl.ds(0, 16)] = sv
        wdst[pl.ds(0, 16)] = dv
        return fill - K

    def tile_body(ti, fill):
        base = w * m_per_w + ti * TILE
        pltpu.sync_copy(src_hbm.at[pl.ds(base, TILE)], src_tile)
        pltpu.sync_copy(dst_hbm.at[pl.ds(base, TILE)], dst_tile)

        def vec_body(vi, fill):
            dv = dst_tile[pl.ds(vi * 16, 16)]
            wv = plsc.load_gather(bitmap, [lax.shift_right_logical(dv, 5)])
            bit = lax.shift_right_logical(wv, dv & 31) & 1
            m = bit == 1
            sv = src_tile[pl.ds(vi * 16, 16)]
            mi = m.astype(jnp.int32)
            cs = plsc.cumsum(mi)
            pos = (fill + cs) - mi
            plsc.store_scatter(wsrc, [pos], sv, mask=m)
            plsc.store_scatter(wdst, [pos], dv, mask=m)
            fill = fill + jnp.sum(mi)
            return lax.cond(fill >= K, flush, lambda f: f, fill)

        return lax.fori_loop(0, TILE // 16, vec_body, fill)

    fill = lax.fori_loop(0, n_tiles, tile_body, jnp.int32(0))

    for j in range(K // 16):
        pos = lane + (j * 16)
        pm = pos >= fill
        sv = wsrc[pl.ds(j * 16, 16)]
        dv = wdst[pl.ds(j * 16, 16)]
        wsrc[pl.ds(j * 16, 16)] = jnp.where(pm, lane + s * 16, sv)
        wdst[pl.ds(j * 16, 16)] = jnp.where(pm, lane + (N_PAD - 16), dv)
    flush(fill)

    plsc.subcore_barrier()

    r0 = s * rows_per_sub
    pltpu.sync_copy(racc_sh.at[pl.ds(r0, rows_per_sub)],
                    out_hbm.at[pl.ds(c * NSLOT_PAD + r0, rows_per_sub)])


@functools.cache
def _make_prop_rows(m_pad):
    mesh = plsc.VectorSubcoreMesh(core_axis_name="c", subcore_axis_name="s")
    body = functools.partial(_prop_rows_body, m_pad=m_pad)
    return pl.kernel(
        body,
        mesh=mesh,
        compiler_params=pltpu.CompilerParams(needs_layout_passes=False,
                                             use_tc_tiling_on_sc=False),
        out_type=jax.ShapeDtypeStruct((2 * NSLOT_PAD, EMB), jnp.float32),
        scratch_types=[
            pltpu.VMEM((K + 32,), jnp.int32),
            pltpu.VMEM((K + 32,), jnp.int32),
            pltpu.VMEM((NSTR, 128), jnp.int32),
            pltpu.VMEM((NSTR, 128), jnp.int32),
            pltpu.VMEM((NSTR, 128), jnp.int32),
            pltpu.VMEM((TILE,), jnp.int32),
            pltpu.VMEM((TILE,), jnp.int32),
            pltpu.VMEM((NSTR, 128, EMB), jnp.float32),
            pltpu.VMEM((64, EMB), jnp.float32),
            pltpu.VMEM((BM_WORDS,), jnp.int32),
            pltpu.VMEM_SHARED((NSLOT_PAD, EMB), jnp.float32),
            pltpu.SemaphoreType.DMA,
            pltpu.SemaphoreType.DMA,
            pltpu.SemaphoreType.DMA,
        ],
    )


def _pad_edges32(src, dst):
    m = src.shape[0]
    step = 2 * NSUB * TILE
    m_pad = ((m + step - 1) // step) * step
    pad = m_pad - m
    if pad:
        fill_idx = (jnp.arange(pad, dtype=jnp.int32) % 16) + (N_PAD - 16)
        src = jnp.concatenate([src, fill_idx])
        dst = jnp.concatenate([dst, fill_idx])
    return src, dst


def _propagate_rows(src, dst, t, bm, pos):
    src, dst = _pad_edges32(src, dst)
    out = _make_prop_rows(src.shape[0])(src, dst, t, bm, pos)
    return out[:NSLOT_PAD] + out[NSLOT_PAD:]


def _pad_edges(src, dst):
    m = src.shape[0]
    step = NSUB * TILE
    m_pad = ((m + step - 1) // step) * step
    pad = m_pad - m
    if pad:
        fill_idx = (jnp.arange(pad, dtype=jnp.int32) % 16) + (N_PAD - 16)
        src = jnp.concatenate([src, fill_idx])
        dst = jnp.concatenate([dst, fill_idx])
    return src, dst


def _graph(u, it):
    src = jnp.concatenate([u, it + U_OFF])
    dst = jnp.concatenate([it + U_OFF, u])
    deg = jnp.bincount(src, length=N_NODES).astype(jnp.float32)
    invd = lax.rsqrt(jnp.maximum(deg, 1.0))
    invd = jnp.pad(invd, (0, N_PAD - N_NODES))[:, None]
    return src, dst, invd


def _lightgcn_sc(e0, graph):
    src, dst, invd = graph
    t0 = e0 * invd
    y1 = _propagate(src, dst, t0)
    t1 = y1 * (invd * invd)
    y2 = _propagate(src, dst, t1)
    return (e0 + invd * (y1 + y2)) * (1.0 / (LAYERS + 1))


def _lightgcn_rows(e0, graph, bm, pos, r_nodes, slots):
    src, dst, invd = graph
    t0 = e0 * invd
    y1 = _propagate(src, dst, t0)
    t1 = y1 * (invd * invd)
    y2r = _propagate_rows(src, dst, t1, bm, pos)
    y2_sel = y2r[slots]
    return (e0[r_nodes] + invd[r_nodes] * (y1[r_nodes] + y2_sel)) * (1.0 / (LAYERS + 1))


def _score_rows(rows):
    u, p, n = rows[:B], rows[B:2 * B], rows[2 * B:]
    return (jax.nn.relu(jnp.sum(u * p, axis=-1)),
            jax.nn.relu(jnp.sum(u * n, axis=-1)))


def _loss_body(fp, fn, ap0, an0, ap1, an1, reg, out):
    def bpr(p, n):
        d = p - n
        sig = 1.0 / (1.0 + jnp.exp(-d))
        return -jnp.mean(jnp.log(sig + 1e-10))

    rec = bpr(fp[...], fn[...])
    aux = 0.5 * (bpr(ap0[...], an0[...]) + bpr(ap1[...], an1[...]))
    val = rec + LAMB * aux + REG * reg[0, 0]
    out[...] = jnp.full((1, 1), val, dtype=jnp.float32)


def _final_loss(fp, fn, ap0, an0, ap1, an1, emb_reg):
    shape = (32, 128)
    args = [x.reshape(shape) for x in (fp, fn, ap0, an0, ap1, an1)]
    args.append(emb_reg.reshape(1, 1))
    out = pl.pallas_call(
        _loss_body,
        out_shape=jax.ShapeDtypeStruct((1, 1), jnp.float32),
    )(*args)
    return out.reshape(())


def _prob(e, u, it):
    return jax.nn.relu(jnp.sum(e[u] * e[U_OFF + it], axis=-1))


def kernel(batch_data, edge_index_aux0, edge_index_aux1, edge_index_target, user_emb, item_emb):
    user = batch_data[:, 0, 0]
    p_item = batch_data[:, 0, 1]
    n_item = batch_data[:, 0, 2]
    all_emb = jnp.concatenate([user_emb, item_emb], axis=0)
    e0 = jnp.pad(all_emb, ((0, N_PAD - N_NODES), (0, 0)))
    aux_edges = [edge_index_aux0, edge_index_aux1]
    tgt_graph = _graph(edge_index_target[0], edge_index_target[1])
    cond_p, cond_n, comb_p, comb_n, aux_p, aux_n = [], [], [], [], [], []
    aux_dots = []
    r_nodes = jnp.concatenate([user, U_OFF + p_item, U_OFF + n_item]).astype(jnp.int32)
    pos_arr = jnp.full((N_PAD,), NSLOT_PAD - 1, jnp.int32)
    pos_arr = pos_arr.at[r_nodes].set(jnp.arange(NSLOT, dtype=jnp.int32))
    pos_arr = pos_arr.at[N_PAD - 16 + jnp.arange(16)].set(
        NSLOT + jnp.arange(16, dtype=jnp.int32))
    slots = pos_arr[r_nodes]
    bmb = jnp.zeros((N_PAD,), jnp.int32).at[r_nodes].set(1)
    bm = jnp.sum(bmb.reshape(BM_WORDS, 32) << jnp.arange(32, dtype=jnp.int32),
                 axis=1, dtype=jnp.int32)
    for idx in range(N_BEH - 1):
        comb_u = jnp.concatenate([aux_edges[idx][0], edge_index_target[0]])
        comb_it = jnp.concatenate([aux_edges[idx][1], edge_index_target[1]])
        comb_rows = _lightgcn_rows(e0, _graph(comb_u, comb_it), bm, pos_arr,
                                   r_nodes, slots)
        cp, cn = _score_rows(comb_rows)
        comb_p.append(cp)
        comb_n.append(cn)
        aux_e = _lightgcn_sc(e0, _graph(aux_edges[idx][0], aux_edges[idx][1]))
        aux_p.append(_prob(aux_e, user, p_item))
        aux_n.append(_prob(aux_e, user, n_item))
        src_t, dst_t, invd_t = tgt_graph
        t0c = aux_e * invd_t
        y1c = _propagate(src_t, dst_t, t0c)
        t1c = y1c * (invd_t * invd_t)
        y2cr = _propagate_rows(src_t, dst_t, t1c, bm, pos_arr)
        cond_rows = (aux_e[r_nodes] + invd_t[r_nodes] * (y1c[r_nodes] + y2cr[slots])) * (1.0 / (LAYERS + 1))
        cp, cn = _score_rows(cond_rows)
        cond_p.append(cp)
        cond_n.append(cn)
        au = aux_e[batch_data[:, 1 + idx, 0]]
        ap = aux_e[U_OFF + batch_data[:, 1 + idx, 1]]
        an = aux_e[U_OFF + batch_data[:, 1 + idx, 2]]
        aux_dots.append((jnp.sum(au * ap, axis=-1), jnp.sum(au * an, axis=-1)))
    cond_p = jnp.stack(cond_p)
    cond_n = jnp.stack(cond_n)
    comb_p = jnp.stack(comb_p)
    comb_n = jnp.stack(comb_n)
    aux_p = jnp.stack(aux_p)
    aux_n = jnp.stack(aux_n)

    def front(cond, comb, aux):
        return jnp.sum(cond * jnp.sum(comb * aux, axis=0), axis=0)

    fp = front(cond_p, comb_p, aux_p)
    fn = front(cond_n, comb_n, aux_n)
    emb_reg = (jnp.linalg.norm(user_emb) + jnp.linalg.norm(item_emb)) / item_emb.shape[0]
    return _final_loss(fp, fn, aux_dots[0][0], aux_dots[0][1], aux_dots[1][0], aux_dots[1][1], emb_reg)


# speedup vs baseline: 16.9747x; 1.2536x over previous
import functools
import jax
import jax.numpy as jnp
from jax import lax
from jax.experimental import pallas as pl
from jax.experimental.pallas import tpu as pltpu
from jax.experimental.pallas import tpu_sc as plsc

N_USERS = 100000
N_ITEMS = 100000
EMB = 64
LAYERS = 2
N_BEH = 3
LAMB = 0.5
REG = 1e-4
B = 4096
E = 500000
U_OFF = N_USERS + 1
N_NODES = (N_USERS + 1) + (N_ITEMS + 1)

N_SUPER = 5
C_CORE = 20480
C_TOTAL = 2 * C_CORE
N_PAD = N_SUPER * C_TOTAL
TILE = 2048
K = 512
NSTR = 4
NSUB = 16
DUMP = C_CORE


def _prop_body(srca_hbm, dsta_hbm, srcb_hbm, dstb_hbm, t_hbm, out_hbm,
               wsrc, wldst, src_dma, ldst_dma, src_tile, dst_tile,
               rows, zbuf, acc_sh, gsem, ssem, *, m_half):
    c = lax.axis_index("c")
    s = lax.axis_index("s")
    m_per_sub = m_half // NSUB
    n_tiles = m_per_sub // TILE
    rows_per_sub = C_CORE // NSUB
    zrows = 64

    zvec = jnp.zeros((16,), jnp.float32)

    def zb(i, _):
        for l in range(EMB // 16):
            zbuf[i, pl.ds(l * 16, 16)] = zvec
        return 0
    lax.fori_loop(0, zrows, zb, 0)

    lane = lax.iota(jnp.int32, 16)

    item_chunks = [i for i in range(N_SUPER)
                   if (i + 1) * C_TOTAL > U_OFF and i * C_TOTAL < N_NODES]
    user_chunks = [i for i in range(N_SUPER) if i * C_TOTAL < U_OFF]

    for sc_i in range(N_SUPER):
        lo = sc_i * C_TOTAL + c * C_CORE

        for z in range(rows_per_sub // zrows):
            pltpu.sync_copy(zbuf, acc_sh.at[pl.ds(s * rows_per_sub + z * zrows, zrows)])

        @pl.when(s == 0)
        def _():
            pltpu.sync_copy(zbuf.at[pl.ds(0, 16)], acc_sh.at[pl.ds(C_CORE, 16)])

        plsc.subcore_barrier()

        def flush(fill):
            for j in range(NSTR):
                for i in range(128 // 16):
                    src_dma[j, pl.ds(i * 16, 16)] = wsrc[pl.ds(j * 128 + i * 16, 16)]
                    ldst_dma[j, pl.ds(i * 16, 16)] = wldst[pl.ds(j * 128 + i * 16, 16)]
            gh = [pltpu.async_copy(t_hbm.at[src_dma.at[j]], rows.at[j], gsem)
                  for j in range(NSTR)]
            sh = []
            for j in range(NSTR):
                gh[j].wait()
                sh.append(pltpu.async_copy(rows.at[j], acc_sh.at[ldst_dma.at[j]],
                                           ssem, add=True))
            for h in sh:
                h.wait()
            sv = wsrc[pl.ds(K, 16)]
            dv = wldst[pl.ds(K, 16)]
            wsrc[pl.ds(0, 16)] = sv
            wldst[pl.ds(0, 16)] = dv
            return fill - K

        def make_tile_body(sh_hbm, dh_hbm):
            def tile_body(ti, fill):
                base = s * m_per_sub + ti * TILE
                pltpu.sync_copy(sh_hbm.at[pl.ds(base, TILE)], src_tile)
                pltpu.sync_copy(dh_hbm.at[pl.ds(base, TILE)], dst_tile)

                def vec_body(vi, fill):
                    dv = dst_tile[pl.ds(vi * 16, 16)]
                    m = (dv >= lo) & (dv < lo + C_CORE)
                    sv = src_tile[pl.ds(vi * 16, 16)]
                    mi = m.astype(jnp.int32)
                    cs = plsc.cumsum(mi)
                    pos = (fill + cs) - mi
                    plsc.store_scatter(wsrc, [pos], sv, mask=m)
                    plsc.store_scatter(wldst, [pos], dv - lo, mask=m)
                    fill = fill + jnp.sum(mi)
                    return lax.cond(fill >= K, flush, lambda f: f, fill)

                return lax.fori_loop(0, TILE // 16, vec_body, fill)
            return tile_body

        fill = jnp.int32(0)
        if sc_i in item_chunks:
            fill = lax.fori_loop(0, n_tiles, make_tile_body(srca_hbm, dsta_hbm), fill)
        if sc_i in user_chunks:
            fill = lax.fori_loop(0, n_tiles, make_tile_body(srcb_hbm, dstb_hbm), fill)

        for j in range(K // 16):
            pos = lane + (j * 16)
            pm = pos >= fill
            sv = wsrc[pl.ds(j * 16, 16)]
            dv = wldst[pl.ds(j * 16, 16)]
            wsrc[pl.ds(j * 16, 16)] = jnp.where(pm, lane + s * 16, sv)
            wldst[pl.ds(j * 16, 16)] = jnp.where(pm, lane + DUMP, dv)
        flush(fill)

        plsc.subcore_barrier()

        for z in range(rows_per_sub // zrows):
            r0 = s * rows_per_sub + z * zrows
            pltpu.sync_copy(acc_sh.at[pl.ds(r0, zrows)],
                            out_hbm.at[pl.ds(lo + r0, zrows)])


@functools.cache
def _make_propagate(m_half):
    mesh = plsc.VectorSubcoreMesh(core_axis_name="c", subcore_axis_name="s")
    body = functools.partial(_prop_body, m_half=m_half)
    return pl.kernel(
        body,
        mesh=mesh,
        compiler_params=pltpu.CompilerParams(needs_layout_passes=False, use_tc_tiling_on_sc=False),
        out_type=jax.ShapeDtypeStruct((N_PAD, EMB), jnp.float32),
        scratch_types=[
            pltpu.VMEM((K + 32,), jnp.int32),
            pltpu.VMEM((K + 32,), jnp.int32),
            pltpu.VMEM((NSTR, 128), jnp.int32),
            pltpu.VMEM((NSTR, 128), jnp.int32),
            pltpu.VMEM((TILE,), jnp.int32),
            pltpu.VMEM((TILE,), jnp.int32),
            pltpu.VMEM((NSTR, 128, EMB), jnp.float32),
            pltpu.VMEM((64, EMB), jnp.float32),
            pltpu.VMEM_SHARED((C_CORE + 16, EMB), jnp.float32),
            pltpu.SemaphoreType.DMA,
            pltpu.SemaphoreType.DMA,
        ],
    )


def _pad_half(x, fill_vals, m_pad):
    pad = m_pad - x.shape[0]
    if pad:
        x = jnp.concatenate([x, fill_vals[:pad]])
    return x


def _propagate(halves, t):
    u, it = halves
    m = u.shape[0]
    step = NSUB * TILE
    m_pad = ((m + step - 1) // step) * step
    zsrc = (jnp.arange(m_pad - m, dtype=jnp.int32) % 16) + (N_PAD - 16)
    srca = _pad_half(u, zsrc, m_pad)
    dsta = _pad_half(it, (jnp.arange(m_pad - m, dtype=jnp.int32) % 16) + U_OFF, m_pad)
    srcb = _pad_half(it, zsrc, m_pad)
    dstb = _pad_half(u, jnp.arange(m_pad - m, dtype=jnp.int32) % 16, m_pad)
    return _make_propagate(m_pad)(srca, dsta, srcb, dstb, t)


NSLOT = 3 * B
NSLOT_PAD = 13312
BM_WORDS = N_PAD // 32


def _prop_rows_body(src_hbm, dst_hbm, t_hbm, bm_hbm, pos_hbm, out_hbm,
                    wsrc, wdst, src_dma, dst_dma, posval, src_tile, dst_tile,
                    rows, zbuf, bitmap, racc_sh, gsem, psem, ssem, *, m_pad):
    c = lax.axis_index("c")
    s = lax.axis_index("s")
    w = c * NSUB + s
    m_per_w = m_pad // (2 * NSUB)
    n_tiles = m_per_w // TILE
    rows_per_sub = NSLOT_PAD // NSUB

    zvec = jnp.zeros((16,), jnp.float32)

    def zb(i, _):
        for l in range(EMB // 16):
            zbuf[i, pl.ds(l * 16, 16)] = zvec
        return 0
    lax.fori_loop(0, 64, zb, 0)

    pltpu.sync_copy(bm_hbm, bitmap)
    for z in range(rows_per_sub // 64):
        pltpu.sync_copy(zbuf, racc_sh.at[pl.ds(s * rows_per_sub + z * 64, 64)])
    plsc.subcore_barrier()

    lane = lax.iota(jnp.int32, 16)

    def flush(fill):
        for j in range(NSTR):
            for i in range(128 // 16):
                src_dma[j, pl.ds(i * 16, 16)] = wsrc[pl.ds(j * 128 + i * 16, 16)]
                dst_dma[j, pl.ds(i * 16, 16)] = wdst[pl.ds(j * 128 + i * 16, 16)]
        gh = [pltpu.async_copy(t_hbm.at[src_dma.at[j]], rows.at[j], gsem)
              for j in range(NSTR)]
        ph = [pltpu.async_copy(pos_hbm.at[dst_dma.at[j]], posval.at[j], psem)
              for j in range(NSTR)]
        sh = []
        for j in range(NSTR):
            gh[j].wait()
            ph[j].wait()
            sh.append(pltpu.async_copy(rows.at[j], racc_sh.at[posval.at[j]],
                                       ssem, add=True))
        for h in sh:
            h.wait()
        sv = wsrc[pl.ds(K, 16)]
        dv = wdst[pl.ds(K, 16)]
        wsrc[pl.ds(0, 16)] = sv
        wdst[pl.ds(0, 16)] = dv
        return fill - K

    def tile_body(ti, fill):
        base = w * m_per_w + ti * TILE
        pltpu.sync_copy(src_hbm.at[pl.ds(base, TILE)], src_tile)
        pltpu.sync_copy(dst_hbm.at[pl.ds(base, TILE)], dst_tile)

        def vec_body(vi, fill):
            dv = dst_tile[pl.ds(vi * 16, 16)]
            wv = plsc.load_gather(bitmap, [lax.shift_right_logical(dv, 5)])
            bit = lax.shift_right_logical(wv, dv & 31) & 1
            m = bit == 1
            sv = src_tile[pl.ds(vi * 16, 16)]
            mi = m.astype(jnp.int32)
            cs = plsc.cumsum(mi)
            pos = (fill + cs) - mi
            plsc.store_scatter(wsrc, [pos], sv, mask=m)
            plsc.store_scatter(wdst, [pos], dv, mask=m)
            fill = fill + jnp.sum(mi)
            return lax.cond(fill >= K, flush, lambda f: f, fill)

        return lax.fori_loop(0, TILE // 16, vec_body, fill)

    fill = lax.fori_loop(0, n_tiles, tile_body, jnp.int32(0))

    for j in range(K // 16):
        pos = lane + (j * 16)
        pm = pos >= fill
        sv = wsrc[pl.ds(j * 16, 16)]
        dv = wdst[pl.ds(j * 16, 16)]
        wsrc[pl.ds(j * 16, 16)] = jnp.where(pm, lane + s * 16, sv)
        wdst[pl.ds(j * 16, 16)] = jnp.where(pm, lane + (N_PAD - 16), dv)
    flush(fill)

    plsc.subcore_barrier()

    r0 = s * rows_per_sub
    pltpu.sync_copy(racc_sh.at[pl.ds(r0, rows_per_sub)],
                    out_hbm.at[pl.ds(c * NSLOT_PAD + r0, rows_per_sub)])


@functools.cache
def _make_prop_rows(m_pad):
    mesh = plsc.VectorSubcoreMesh(core_axis_name="c", subcore_axis_name="s")
    body = functools.partial(_prop_rows_body, m_pad=m_pad)
    return pl.kernel(
        body,
        mesh=mesh,
        compiler_params=pltpu.CompilerParams(needs_layout_passes=False,
                                             use_tc_tiling_on_sc=False),
        out_type=jax.ShapeDtypeStruct((2 * NSLOT_PAD, EMB), jnp.float32),
        scratch_types=[
            pltpu.VMEM((K + 32,), jnp.int32),
            pltpu.VMEM((K + 32,), jnp.int32),
            pltpu.VMEM((NSTR, 128), jnp.int32),
            pltpu.VMEM((NSTR, 128), jnp.int32),
            pltpu.VMEM((NSTR, 128), jnp.int32),
            pltpu.VMEM((TILE,), jnp.int32),
            pltpu.VMEM((TILE,), jnp.int32),
            pltpu.VMEM((NSTR, 128, EMB), jnp.float32),
            pltpu.VMEM((64, EMB), jnp.float32),
            pltpu.VMEM((BM_WORDS,), jnp.int32),
            pltpu.VMEM_SHARED((NSLOT_PAD, EMB), jnp.float32),
            pltpu.SemaphoreType.DMA,
            pltpu.SemaphoreType.DMA,
            pltpu.SemaphoreType.DMA,
        ],
    )


def _pad_edges32(src, dst):
    m = src.shape[0]
    step = 2 * NSUB * TILE
    m_pad = ((m + step - 1) // step) * step
    pad = m_pad - m
    if pad:
        fill_idx = (jnp.arange(pad, dtype=jnp.int32) % 16) + (N_PAD - 16)
        src = jnp.concatenate([src, fill_idx])
        dst = jnp.concatenate([dst, fill_idx])
    return src, dst


def _propagate_rows(halves, t, bm, pos):
    u, itn = halves
    src = jnp.concatenate([u, itn])
    dst = jnp.concatenate([itn, u])
    src, dst = _pad_edges32(src, dst)
    out = _make_prop_rows(src.shape[0])(src, dst, t, bm, pos)
    return out[:NSLOT_PAD] + out[NSLOT_PAD:]


def _graph(u, it):
    itn = it + U_OFF
    deg = (jnp.bincount(u, length=N_NODES)
           + jnp.bincount(itn, length=N_NODES)).astype(jnp.float32)
    invd = lax.rsqrt(jnp.maximum(deg, 1.0))
    invd = jnp.pad(invd, (0, N_PAD - N_NODES))[:, None]
    return u, itn, invd


def _lightgcn_sc(e0, graph):
    u, itn, invd = graph
    t0 = e0 * invd
    y1 = _propagate((u, itn), t0)
    t1 = y1 * (invd * invd)
    y2 = _propagate((u, itn), t1)
    return (e0 + invd * (y1 + y2)) * (1.0 / (LAYERS + 1))


def _lightgcn_rows(e0, graph, bm, pos, r_nodes, slots):
    u, itn, invd = graph
    t0 = e0 * invd
    y1 = _propagate((u, itn), t0)
    t1 = y1 * (invd * invd)
    y2r = _propagate_rows((u, itn), t1, bm, pos)
    y2_sel = y2r[slots]
    return (e0[r_nodes] + invd[r_nodes] * (y1[r_nodes] + y2_sel)) * (1.0 / (LAYERS + 1))


def _score_rows(rows):
    u, p, n = rows[:B], rows[B:2 * B], rows[2 * B:]
    return (jax.nn.relu(jnp.sum(u * p, axis=-1)),
            jax.nn.relu(jnp.sum(u * n, axis=-1)))


def _loss_body(fp, fn, ap0, an0, ap1, an1, reg, out):
    def bpr(p, n):
        d = p - n
        sig = 1.0 / (1.0 + jnp.exp(-d))
        return -jnp.mean(jnp.log(sig + 1e-10))

    rec = bpr(fp[...], fn[...])
    aux = 0.5 * (bpr(ap0[...], an0[...]) + bpr(ap1[...], an1[...]))
    val = rec + LAMB * aux + REG * reg[0, 0]
    out[...] = jnp.full((1, 1), val, dtype=jnp.float32)


def _final_loss(fp, fn, ap0, an0, ap1, an1, emb_reg):
    shape = (32, 128)
    args = [x.reshape(shape) for x in (fp, fn, ap0, an0, ap1, an1)]
    args.append(emb_reg.reshape(1, 1))
    out = pl.pallas_call(
        _loss_body,
        out_shape=jax.ShapeDtypeStruct((1, 1), jnp.float32),
    )(*args)
    return out.reshape(())


def _prob(e, u, it):
    return jax.nn.relu(jnp.sum(e[u] * e[U_OFF + it], axis=-1))


def kernel(batch_data, edge_index_aux0, edge_index_aux1, edge_index_target, user_emb, item_emb):
    user = batch_data[:, 0, 0]
    p_item = batch_data[:, 0, 1]
    n_item = batch_data[:, 0, 2]
    all_emb = jnp.concatenate([user_emb, item_emb], axis=0)
    e0 = jnp.pad(all_emb, ((0, N_PAD - N_NODES), (0, 0)))
    aux_edges = [edge_index_aux0, edge_index_aux1]
    tgt_graph = _graph(edge_index_target[0], edge_index_target[1])
    cond_p, cond_n, comb_p, comb_n, aux_p, aux_n = [], [], [], [], [], []
    aux_dots = []
    r_nodes = jnp.concatenate([user, U_OFF + p_item, U_OFF + n_item]).astype(jnp.int32)
    pos_arr = jnp.full((N_PAD,), NSLOT_PAD - 1, jnp.int32)
    pos_arr = pos_arr.at[r_nodes].set(jnp.arange(NSLOT, dtype=jnp.int32))
    pos_arr = pos_arr.at[N_PAD - 16 + jnp.arange(16)].set(
        NSLOT + jnp.arange(16, dtype=jnp.int32))
    slots = pos_arr[r_nodes]
    bmb = jnp.zeros((N_PAD,), jnp.int32).at[r_nodes].set(1)
    bm = jnp.sum(bmb.reshape(BM_WORDS, 32) << jnp.arange(32, dtype=jnp.int32),
                 axis=1, dtype=jnp.int32)
    for idx in range(N_BEH - 1):
        comb_u = jnp.concatenate([aux_edges[idx][0], edge_index_target[0]])
        comb_it = jnp.concatenate([aux_edges[idx][1], edge_index_target[1]])
        comb_rows = _lightgcn_rows(e0, _graph(comb_u, comb_it), bm, pos_arr,
                                   r_nodes, slots)
        cp, cn = _score_rows(comb_rows)
        comb_p.append(cp)
        comb_n.append(cn)
        aux_e = _lightgcn_sc(e0, _graph(aux_edges[idx][0], aux_edges[idx][1]))
        aux_p.append(_prob(aux_e, user, p_item))
        aux_n.append(_prob(aux_e, user, n_item))
        u_t, itn_t, invd_t = tgt_graph
        t0c = aux_e * invd_t
        y1c = _propagate((u_t, itn_t), t0c)
        t1c = y1c * (invd_t * invd_t)
        y2cr = _propagate_rows((u_t, itn_t), t1c, bm, pos_arr)
        cond_rows = (aux_e[r_nodes] + invd_t[r_nodes] * (y1c[r_nodes] + y2cr[slots])) * (1.0 / (LAYERS + 1))
        cp, cn = _score_rows(cond_rows)
        cond_p.append(cp)
        cond_n.append(cn)
        au = aux_e[batch_data[:, 1 + idx, 0]]
        ap = aux_e[U_OFF + batch_data[:, 1 + idx, 1]]
        an = aux_e[U_OFF + batch_data[:, 1 + idx, 2]]
        aux_dots.append((jnp.sum(au * ap, axis=-1), jnp.sum(au * an, axis=-1)))
    cond_p = jnp.stack(cond_p)
    cond_n = jnp.stack(cond_n)
    comb_p = jnp.stack(comb_p)
    comb_n = jnp.stack(comb_n)
    aux_p = jnp.stack(aux_p)
    aux_n = jnp.stack(aux_n)

    def front(cond, comb, aux):
        return jnp.sum(cond * jnp.sum(comb * aux, axis=0), axis=0)

    fp = front(cond_p, comb_p, aux_p)
    fn = front(cond_n, comb_n, aux_n)
    emb_reg = (jnp.linalg.norm(user_emb) + jnp.linalg.norm(item_emb)) / item_emb.shape[0]
    return _final_loss(fp, fn, aux_dots[0][0], aux_dots[0][1], aux_dots[1][0], aux_dots[1][1], emb_reg)
